# baseline (speedup 1.0000x reference)
import jax
import jax.numpy as jnp
from jax import lax
from jax.experimental import pallas as pl
from jax.experimental.pallas import tpu as pltpu
from jax.experimental.pallas import tpu_sc as plsc

_BATCH = 4
_SEQ = 8192
_HIDDEN = 128
_LANES = 16
_NC = 2
_NS = 16
_NW = _NC * _NS
_SEQ_W = _SEQ // _NW
_CHUNK = 128
_NCHUNK = _SEQ_W // _CHUNK
_NITEM = _BATCH * _NCHUNK
_NBUF = 5


def _body(
    ids_hbm, tok_hbm, pos_hbm, out_hbm, pout_hbm,
    idx_v, pos_v, pid_v, bufs_v, gsems, osems, isem, psem,
):
    wid = lax.axis_index("s") * _NC + lax.axis_index("c")
    seq0 = wid * _SEQ_W
    bufs = [bufs_v.at[i] for i in range(_NBUF)]

    icopy = pltpu.async_copy(
        ids_hbm.at[:, pl.ds(seq0, _SEQ_W)], idx_v, isem
    )
    pcopy = pltpu.async_copy(pos_hbm.at[pl.ds(seq0, _SEQ_W)], pos_v, psem)

    @plsc.parallel_loop(0, _SEQ_W // _LANES, 1, unroll=4)
    def _mk_pos(r):
        val = lax.iota(jnp.int32, _LANES) + (seq0 + r * _LANES)
        for b in range(_BATCH):
            pid_v[b, pl.ds(r * _LANES, _LANES)] = val

    pstore = pltpu.async_copy(
        pid_v, pout_hbm.at[:, pl.ds(seq0, _SEQ_W)], psem
    )

    def fire_gather(t):
        b, j = divmod(t, _NCHUNK)
        return pltpu.async_copy(
            tok_hbm.at[idx_v.at[b, pl.ds(j * _CHUNK, _CHUNK)]],
            bufs[t % _NBUF],
            gsems.at[t % _NBUF],
        )

    icopy.wait()

    gathers = [None] * _NITEM
    stores = [None] * _NITEM
    for t in range(_NBUF - 1):
        gathers[t] = fire_gather(t)
    pcopy.wait()

    for t in range(_NITEM):
        b, j = divmod(t, _NCHUNK)
        gathers[t].wait()
        buf = bufs[t % _NBUF]
        prow = j * _CHUNK

        @plsc.parallel_loop(0, _CHUNK, 1, unroll=16)
        def _add_row(r):
            for v in range(_HIDDEN // _LANES):
                x = pos_v[prow + r, pl.ds(v * _LANES, _LANES)]
                plsc.addupdate(buf.at[r, pl.ds(v * _LANES, _LANES)], x)

        stores[t] = pltpu.async_copy(
            buf,
            out_hbm.at[b, pl.ds(seq0 + j * _CHUNK, _CHUNK)],
            osems.at[t % _NBUF],
        )
        nxt = t + _NBUF - 1
        if nxt < _NITEM:
            if nxt - _NBUF >= 0:
                stores[nxt - _NBUF].wait()
            gathers[nxt] = fire_gather(nxt)

    for t in range(_NITEM - _NBUF, _NITEM):
        stores[t].wait()
    pstore.wait()


def kernel(input_ids, tok_emb, pos_emb):
    k = pl.kernel(
        _body,
        out_type=(
            jax.ShapeDtypeStruct((_BATCH, _SEQ, _HIDDEN), jnp.float32),
            jax.ShapeDtypeStruct((_BATCH, _SEQ), jnp.int32),
        ),
        mesh=plsc.VectorSubcoreMesh(core_axis_name="c", subcore_axis_name="s"),
        scratch_types=[
            pltpu.VMEM((_BATCH, _SEQ_W), jnp.int32),
            pltpu.VMEM((_SEQ_W, _HIDDEN), jnp.float32),
            pltpu.VMEM((_BATCH, _SEQ_W), jnp.int32),
            pltpu.VMEM((_NBUF, _CHUNK, _HIDDEN), jnp.float32),
            pltpu.SemaphoreType.DMA((_NBUF,)),
            pltpu.SemaphoreType.DMA((_NBUF,)),
            pltpu.SemaphoreType.DMA,
            pltpu.SemaphoreType.DMA,
        ],
    )
    x, pos = k(input_ids.astype(jnp.int32), tok_emb, pos_emb)
    return (x, pos.astype(input_ids.dtype))


# speedup vs baseline: 1.4333x; 1.4333x over previous
import jax
import jax.numpy as jnp
from jax import lax
from jax.experimental import pallas as pl
from jax.experimental.pallas import tpu as pltpu
from jax.experimental.pallas import tpu_sc as plsc

_BATCH = 4
_SEQ = 8192
_HIDDEN = 128
_LANES = 16
_NC = 2
_NS = 16
_NW = _NC * _NS
_SEQ_W = _SEQ // _NW
_CHUNK = 128
_NCHUNK = _SEQ_W // _CHUNK
_NITEM = _BATCH * _NCHUNK
_NBUF = 5


def _body(
    ids_hbm, tok_hbm, pos_hbm, out_hbm, pout_hbm,
    idx_v, pos_v, pid_v, bufs_v, gsems, osems, isem, psem,
):
    wid = lax.axis_index("s") * _NC + lax.axis_index("c")
    seq0 = wid * _SEQ_W
    bufs = [bufs_v.at[i] for i in range(_NBUF)]

    icopy = pltpu.async_copy(
        ids_hbm.at[:, pl.ds(seq0, _SEQ_W)], idx_v, isem
    )
    pcopy = pltpu.async_copy(pos_hbm.at[pl.ds(seq0, _SEQ_W)], pos_v, psem)

    @plsc.parallel_loop(0, _SEQ_W // _LANES, 1, unroll=4)
    def _mk_pos(r):
        val = lax.iota(jnp.int32, _LANES) + (seq0 + r * _LANES)
        for b in range(_BATCH):
            pid_v[b, pl.ds(r * _LANES, _LANES)] = val

    pstore = pltpu.async_copy(
        pid_v, pout_hbm.at[:, pl.ds(seq0, _SEQ_W)], psem
    )

    def fire_gather(t):
        j, b = divmod(t, _BATCH)
        return pltpu.async_copy(
            tok_hbm.at[idx_v.at[b, pl.ds(j * _CHUNK, _CHUNK)]],
            bufs[t % _NBUF],
            gsems.at[t % _NBUF],
        )

    icopy.wait()

    gathers = [None] * _NITEM
    stores = [None] * _NITEM
    for t in range(_NBUF - 1):
        gathers[t] = fire_gather(t)
    pcopy.wait()

    for g in range(_NCHUNK):
        grp = [g * _BATCH + b for b in range(_BATCH)]
        for t in grp:
            gathers[t].wait()
        gbufs = [bufs[t % _NBUF] for t in grp]
        prow = g * _CHUNK

        @plsc.parallel_loop(0, _CHUNK, 1, unroll=2)
        def _add_row(r):
            for v in range(_HIDDEN // _LANES):
                x = pos_v[prow + r, pl.ds(v * _LANES, _LANES)]
                for buf in gbufs:
                    plsc.addupdate(buf.at[r, pl.ds(v * _LANES, _LANES)], x)

        for b, t in enumerate(grp):
            stores[t] = pltpu.async_copy(
                gbufs[b],
                out_hbm.at[b, pl.ds(seq0 + g * _CHUNK, _CHUNK)],
                osems.at[t % _NBUF],
            )
        for t in grp:
            nxt = t + _NBUF - 1
            if nxt < _NITEM:
                if nxt - _NBUF >= 0:
                    stores[nxt - _NBUF].wait()
                gathers[nxt] = fire_gather(nxt)

    for t in range(_NITEM - _NBUF, _NITEM):
        stores[t].wait()
    pstore.wait()


def kernel(input_ids, tok_emb, pos_emb):
    k = pl.kernel(
        _body,
        out_type=(
            jax.ShapeDtypeStruct((_BATCH, _SEQ, _HIDDEN), jnp.float32),
            jax.ShapeDtypeStruct((_BATCH, _SEQ), jnp.int32),
        ),
        mesh=plsc.VectorSubcoreMesh(core_axis_name="c", subcore_axis_name="s"),
        scratch_types=[
            pltpu.VMEM((_BATCH, _SEQ_W), jnp.int32),
            pltpu.VMEM((_SEQ_W, _HIDDEN), jnp.float32),
            pltpu.VMEM((_BATCH, _SEQ_W), jnp.int32),
            pltpu.VMEM((_NBUF, _CHUNK, _HIDDEN), jnp.float32),
            pltpu.SemaphoreType.DMA((_NBUF,)),
            pltpu.SemaphoreType.DMA((_NBUF,)),
            pltpu.SemaphoreType.DMA,
            pltpu.SemaphoreType.DMA,
        ],
    )
    x, pos = k(input_ids.astype(jnp.int32), tok_emb, pos_emb)
    return (x, pos.astype(input_ids.dtype))


# speedup vs baseline: 1.4851x; 1.0361x over previous
import jax
import jax.numpy as jnp
from jax import lax
from jax.experimental import pallas as pl
from jax.experimental.pallas import tpu as pltpu
from jax.experimental.pallas import tpu_sc as plsc

_BATCH = 4
_SEQ = 8192
_HIDDEN = 128
_LANES = 16
_NC = 2
_NS = 16
_NW = _NC * _NS
_SEQ_W = _SEQ // _NW
_CHUNK = 128
_NCHUNK = _SEQ_W // _CHUNK
_NITEM = _BATCH * _NCHUNK
_NBUF = 5


def _body(
    ids_hbm, tok_hbm, pos_hbm, out_hbm, pout_hbm,
    idx_v, pos_v, pid_v, bufs_v, gsems, osems, isem, psem,
):
    wid = lax.axis_index("s") * _NC + lax.axis_index("c")
    seq0 = wid * _SEQ_W
    bufs = [bufs_v.at[i] for i in range(_NBUF)]

    icopy = pltpu.async_copy(
        ids_hbm.at[:, pl.ds(seq0, _SEQ_W)], idx_v, isem
    )
    pcopy = pltpu.async_copy(pos_hbm.at[pl.ds(seq0, _SEQ_W)], pos_v, psem)

    @plsc.parallel_loop(0, _SEQ_W // _LANES, 1, unroll=4)
    def _mk_pos(r):
        val = lax.iota(jnp.int32, _LANES) + (seq0 + r * _LANES)
        for b in range(_BATCH):
            pid_v[b, pl.ds(r * _LANES, _LANES)] = val

    pstore = pltpu.async_copy(
        pid_v, pout_hbm.at[:, pl.ds(seq0, _SEQ_W)], psem
    )

    def fire_gather(t):
        j, b = divmod(t, _BATCH)
        return pltpu.async_copy(
            tok_hbm.at[idx_v.at[b, pl.ds(j * _CHUNK, _CHUNK)]],
            bufs[t % _NBUF],
            gsems.at[t % _NBUF],
        )

    icopy.wait()

    gathers = [None] * _NITEM
    stores = [None] * _NITEM
    for t in range(_NBUF - 1):
        gathers[t] = fire_gather(t)
    pcopy.wait()

    for t in range(_NITEM):
        j, b = divmod(t, _BATCH)
        gathers[t].wait()
        buf = bufs[t % _NBUF]
        prow = j * _CHUNK

        @plsc.parallel_loop(0, _CHUNK, 1, unroll=8)
        def _add_row(r):
            for v in range(_HIDDEN // _LANES):
                x = pos_v[prow + r, pl.ds(v * _LANES, _LANES)]
                plsc.addupdate(buf.at[r, pl.ds(v * _LANES, _LANES)], x)

        stores[t] = pltpu.async_copy(
            buf,
            out_hbm.at[b, pl.ds(seq0 + j * _CHUNK, _CHUNK)],
            osems.at[t % _NBUF],
        )
        nxt = t + _NBUF - 1
        if nxt < _NITEM:
            if nxt - _NBUF >= 0:
                stores[nxt - _NBUF].wait()
            gathers[nxt] = fire_gather(nxt)

    for t in range(_NITEM - _NBUF, _NITEM):
        stores[t].wait()
    pstore.wait()


def kernel(input_ids, tok_emb, pos_emb):
    k = pl.kernel(
        _body,
        out_type=(
            jax.ShapeDtypeStruct((_BATCH, _SEQ, _HIDDEN), jnp.float32),
            jax.ShapeDtypeStruct((_BATCH, _SEQ), jnp.int32),
        ),
        mesh=plsc.VectorSubcoreMesh(core_axis_name="c", subcore_axis_name="s"),
        scratch_types=[
            pltpu.VMEM((_BATCH, _SEQ_W), jnp.int32),
            pltpu.VMEM((_SEQ_W, _HIDDEN), jnp.float32),
            pltpu.VMEM((_BATCH, _SEQ_W), jnp.int32),
            pltpu.VMEM((_NBUF, _CHUNK, _HIDDEN), jnp.float32),
            pltpu.SemaphoreType.DMA((_NBUF,)),
            pltpu.SemaphoreType.DMA((_NBUF,)),
            pltpu.SemaphoreType.DMA,
            pltpu.SemaphoreType.DMA,
        ],
    )
    x, pos = k(input_ids.astype(jnp.int32), tok_emb, pos_emb)
    return (x, pos.astype(input_ids.dtype))


# speedup vs baseline: 1.5782x; 1.0627x over previous
import jax
import jax.numpy as jnp
from jax import lax
from jax.experimental import pallas as pl
from jax.experimental.pallas import tpu as pltpu
from jax.experimental.pallas import tpu_sc as plsc

_BATCH = 4
_SEQ = 8192
_HIDDEN = 128
_LANES = 16
_NC = 2
_NS = 16
_NW = _NC * _NS
_SEQ_W = _SEQ // _NW
_CHUNK = 128
_NCHUNK = _SEQ_W // _CHUNK
_NITEM = _BATCH * _NCHUNK
_NBUF = 5


def _body(
    ids_hbm, tok_hbm, pos_hbm, out_hbm, pout_hbm,
    idx_v, pos_v, pid_v, bufs_v, gsems, osems, isem, psem,
):
    wid = lax.axis_index("s") * _NC + lax.axis_index("c")
    seq0 = wid * _SEQ_W
    bufs = [bufs_v.at[i] for i in range(_NBUF)]

    icopy = pltpu.async_copy(
        ids_hbm.at[:, pl.ds(seq0, _SEQ_W)], idx_v, isem
    )
    pcopy = pltpu.async_copy(pos_hbm.at[pl.ds(seq0, _SEQ_W)], pos_v, psem)

    @plsc.parallel_loop(0, _SEQ_W // _LANES, 1, unroll=4)
    def _mk_pos(r):
        val = lax.iota(jnp.int32, _LANES) + (seq0 + r * _LANES)
        for b in range(_BATCH):
            pid_v[b, pl.ds(r * _LANES, _LANES)] = val

    pstore = pltpu.async_copy(
        pid_v, pout_hbm.at[:, pl.ds(seq0, _SEQ_W)], psem
    )

    def fire_gather(t):
        j, b = divmod(t, _BATCH)
        return pltpu.async_copy(
            tok_hbm.at[idx_v.at[b, pl.ds(j * _CHUNK, _CHUNK)]],
            bufs[t % _NBUF],
            gsems.at[t % _NBUF],
        )

    icopy.wait()

    gathers = [None] * _NITEM
    stores = [None] * _NITEM
    for t in range(_NBUF - 1):
        gathers[t] = fire_gather(t)
    pcopy.wait()

    for t in range(_NITEM):
        j, b = divmod(t, _BATCH)
        gathers[t].wait()
        buf = bufs[t % _NBUF]
        prow = j * _CHUNK

        @plsc.parallel_loop(0, _CHUNK, 1, unroll=4)
        def _add_row(r):
            for v in range(_HIDDEN // _LANES):
                x = pos_v[prow + r, pl.ds(v * _LANES, _LANES)]
                plsc.addupdate(buf.at[r, pl.ds(v * _LANES, _LANES)], x)

        stores[t] = pltpu.async_copy(
            buf,
            out_hbm.at[b, pl.ds(seq0 + j * _CHUNK, _CHUNK)],
            osems.at[t % _NBUF],
        )
        nxt = t + _NBUF - 1
        if nxt < _NITEM:
            if nxt - _NBUF >= 0:
                stores[nxt - _NBUF].wait()
            gathers[nxt] = fire_gather(nxt)

    for t in range(_NITEM - _NBUF, _NITEM):
        stores[t].wait()
    pstore.wait()


def kernel(input_ids, tok_emb, pos_emb):
    k = pl.kernel(
        _body,
        out_type=(
            jax.ShapeDtypeStruct((_BATCH, _SEQ, _HIDDEN), jnp.float32),
            jax.ShapeDtypeStruct((_BATCH, _SEQ), jnp.int32),
        ),
        mesh=plsc.VectorSubcoreMesh(core_axis_name="c", subcore_axis_name="s"),
        scratch_types=[
            pltpu.VMEM((_BATCH, _SEQ_W), jnp.int32),
            pltpu.VMEM((_SEQ_W, _HIDDEN), jnp.float32),
            pltpu.VMEM((_BATCH, _SEQ_W), jnp.int32),
            pltpu.VMEM((_NBUF, _CHUNK, _HIDDEN), jnp.float32),
            pltpu.SemaphoreType.DMA((_NBUF,)),
            pltpu.SemaphoreType.DMA((_NBUF,)),
            pltpu.SemaphoreType.DMA,
            pltpu.SemaphoreType.DMA,
        ],
    )
    x, pos = k(input_ids.astype(jnp.int32), tok_emb, pos_emb)
    return (x, pos.astype(input_ids.dtype))


# speedup vs baseline: 1.6328x; 1.0346x over previous
import jax
import jax.numpy as jnp
from jax import lax
from jax.experimental import pallas as pl
from jax.experimental.pallas import tpu as pltpu
from jax.experimental.pallas import tpu_sc as plsc

_BATCH = 4
_SEQ = 8192
_HIDDEN = 128
_LANES = 16
_NC = 2
_NS = 16
_NW = _NC * _NS
_SEQ_W = _SEQ // _NW
_CHUNK = 128
_NCHUNK = _SEQ_W // _CHUNK
_NITEM = _BATCH * _NCHUNK
_NBUF = 5


def _body(
    ids_hbm, tok_hbm, pos_hbm, out_hbm, pout_hbm,
    idx_v, pos_v, pid_v, bufs_v, gsems, osems, isem, psem,
):
    wid = lax.axis_index("s") * _NC + lax.axis_index("c")
    seq0 = wid * _SEQ_W
    bufs = [bufs_v.at[i] for i in range(_NBUF)]

    icopy = pltpu.async_copy(
        ids_hbm.at[:, pl.ds(seq0, _SEQ_W)], idx_v, isem
    )
    pcopy = pltpu.async_copy(pos_hbm.at[pl.ds(seq0, _SEQ_W)], pos_v, psem)

    @plsc.parallel_loop(0, _SEQ_W // _LANES, 1, unroll=4)
    def _mk_pos(r):
        val = lax.iota(jnp.int32, _LANES) + (seq0 + r * _LANES)
        for b in range(_BATCH):
            pid_v[b, pl.ds(r * _LANES, _LANES)] = val

    pstore = pltpu.async_copy(
        pid_v, pout_hbm.at[:, pl.ds(seq0, _SEQ_W)], psem
    )

    def fire_gather(t):
        j, b = divmod(t, _BATCH)
        return pltpu.async_copy(
            tok_hbm.at[idx_v.at[b, pl.ds(j * _CHUNK, _CHUNK)]],
            bufs[t % _NBUF],
            gsems.at[t % _NBUF],
        )

    icopy.wait()

    gathers = [None] * _NITEM
    stores = [None] * _NITEM
    for t in range(_NBUF - 1):
        gathers[t] = fire_gather(t)
    pcopy.wait()

    for t in range(_NITEM):
        j, b = divmod(t, _BATCH)
        gathers[t].wait()
        buf = bufs[t % _NBUF]
        prow = j * _CHUNK

        @plsc.parallel_loop(0, _CHUNK, 1, unroll=2)
        def _add_row(r):
            for v in range(_HIDDEN // _LANES):
                x = pos_v[prow + r, pl.ds(v * _LANES, _LANES)]
                plsc.addupdate(buf.at[r, pl.ds(v * _LANES, _LANES)], x)

        stores[t] = pltpu.async_copy(
            buf,
            out_hbm.at[b, pl.ds(seq0 + j * _CHUNK, _CHUNK)],
            osems.at[t % _NBUF],
        )
        nxt = t + _NBUF - 1
        if nxt < _NITEM:
            if nxt - _NBUF >= 0:
                stores[nxt - _NBUF].wait()
            gathers[nxt] = fire_gather(nxt)

    for t in range(_NITEM - _NBUF, _NITEM):
        stores[t].wait()
    pstore.wait()


def kernel(input_ids, tok_emb, pos_emb):
    k = pl.kernel(
        _body,
        out_type=(
            jax.ShapeDtypeStruct((_BATCH, _SEQ, _HIDDEN), jnp.float32),
            jax.ShapeDtypeStruct((_BATCH, _SEQ), jnp.int32),
        ),
        mesh=plsc.VectorSubcoreMesh(core_axis_name="c", subcore_axis_name="s"),
        scratch_types=[
            pltpu.VMEM((_BATCH, _SEQ_W), jnp.int32),
            pltpu.VMEM((_SEQ_W, _HIDDEN), jnp.float32),
            pltpu.VMEM((_BATCH, _SEQ_W), jnp.int32),
            pltpu.VMEM((_NBUF, _CHUNK, _HIDDEN), jnp.float32),
            pltpu.SemaphoreType.DMA((_NBUF,)),
            pltpu.SemaphoreType.DMA((_NBUF,)),
            pltpu.SemaphoreType.DMA,
            pltpu.SemaphoreType.DMA,
        ],
    )
    x, pos = k(input_ids.astype(jnp.int32), tok_emb, pos_emb)
    return (x, pos.astype(input_ids.dtype))
